# baseline (speedup 1.0000x reference)
import jax
import jax.numpy as jnp
from jax import lax
from jax.experimental import pallas as pl
from jax.experimental.pallas import tpu as pltpu
from jax.experimental.pallas import tpu_sc as plsc

VOCAB = 1000000
EMBED = 32
B, F, L = 4096, 26, 20

NC, NS = 2, 16
NW = NC * NS

SEGS = B * F
SEGS_PER_W = SEGS // NW
CHUNK_SEGS = 64
ROWS_PER_CHUNK = CHUNK_SEGS * L
GATHER_W = 128
N_GATHERS = ROWS_PER_CHUNK // GATHER_W
N_CHUNKS = SEGS_PER_W // CHUNK_SEGS

VB = 256
NBLK = (VOCAB // 128) // 2
TAIL = VOCAB - NBLK * VB
BLK_PER_W = 122
BLK_ELS = VB * EMBED


def _tr_body(tt_hbm, tail_hbm, tflat_hbm, inb0, inb1, outb0, outb1, tailb,
             isem0, isem1, osem0, osem1):
    inbs = (inb0, inb1)
    outbs = (outb0, outb1)
    isems = (isem0, isem1)
    osems = (osem0, osem1)
    cid = lax.axis_index("c")
    sid = lax.axis_index("s")
    w = sid * NC + cid
    blk0 = w * BLK_PER_W
    cvec = lax.iota(jnp.int32, 16) * EMBED

    def fire_in(i, s):
        pltpu.async_copy(tt_hbm.at[:, pl.ds(i * VB, VB)], inbs[s], isems[s])

    def transpose_block(s):
        def e_body(e, carry):
            ide = cvec + e
            for cc in range(VB // 16):
                val = inbs[s][e, pl.ds(cc * 16, 16)]
                plsc.store_scatter(outbs[s], [ide + cc * 16 * EMBED], val)
            return carry

        lax.fori_loop(0, EMBED, e_body, 0)

    fire_in(blk0, 0)
    fire_in(blk0 + 1, 1)

    def outer(g, carry):
        for s in range(2):
            i = blk0 + 2 * g + s
            pltpu.make_async_copy(
                tt_hbm.at[:, pl.ds(0, VB)], inbs[s], isems[s]).wait()

            @pl.when(g > 0)
            def _():
                pltpu.make_async_copy(
                    outbs[s], tflat_hbm.at[pl.ds(0, BLK_ELS)], osems[s]
                ).wait()

            transpose_block(s)
            pltpu.async_copy(
                outbs[s], tflat_hbm.at[pl.ds(i * BLK_ELS, BLK_ELS)], osems[s]
            )

            @pl.when(2 * g + s + 2 < BLK_PER_W)
            def _():
                fire_in(i + 2, s)

        return carry

    lax.fori_loop(0, BLK_PER_W // 2, outer, 0)
    for s in range(2):
        pltpu.make_async_copy(
            outbs[s], tflat_hbm.at[pl.ds(0, BLK_ELS)], osems[s]).wait()

    @pl.when(w >= NW - 2)
    def _():
        i = NBLK - 2 + (w - (NW - 2))
        pltpu.sync_copy(tt_hbm.at[:, pl.ds(i * VB, VB)], inb0)
        transpose_block(0)
        pltpu.sync_copy(outb0, tflat_hbm.at[pl.ds(i * BLK_ELS, BLK_ELS)])

    @pl.when(w == NW - 1)
    def _():
        pltpu.sync_copy(tail_hbm, tailb)
        pltpu.sync_copy(
            tailb, tflat_hbm.at[pl.ds(NBLK * BLK_ELS, TAIL * EMBED)])


def _gather_body(idx_hbm, table_hbm, out_hbm, idx_v, rows_v, outbuf, sem0, sem1):
    sems = (sem0, sem1)
    cid = lax.axis_index("c")
    sid = lax.axis_index("s")
    w = sid * NC + cid
    idx_base = w * (SEGS_PER_W * L)
    seg_base = w * SEGS_PER_W

    def load_and_fire(c, b):
        pltpu.sync_copy(
            idx_hbm.at[pl.ds(idx_base + c * ROWS_PER_CHUNK, ROWS_PER_CHUNK)],
            idx_v.at[b],
        )
        for j in range(N_GATHERS):
            pltpu.async_copy(
                table_hbm.at[idx_v.at[b, pl.ds(j * GATHER_W, GATHER_W)]],
                rows_v.at[b, pl.ds(j * GATHER_W, GATHER_W)],
                sems[b],
            )

    def drain(b):
        pltpu.make_async_copy(
            table_hbm.at[pl.ds(0, ROWS_PER_CHUNK)], rows_v.at[b], sems[b]
        ).wait()

    def compute(c, b):
        def seg_body(s, carry):
            r0 = s * L
            acc0 = rows_v[b, r0, pl.ds(0, 16)]
            acc1 = rows_v[b, r0, pl.ds(16, 16)]
            for j in range(1, L):
                acc0 = acc0 + rows_v[b, r0 + j, pl.ds(0, 16)]
                acc1 = acc1 + rows_v[b, r0 + j, pl.ds(16, 16)]
            outbuf[b, s, pl.ds(0, 16)] = acc0
            outbuf[b, s, pl.ds(16, 16)] = acc1
            return carry

        lax.fori_loop(0, CHUNK_SEGS, seg_body, 0)
        pltpu.sync_copy(
            outbuf.at[b],
            out_hbm.at[pl.ds(seg_base + c * CHUNK_SEGS, CHUNK_SEGS)],
        )

    load_and_fire(0, 0)
    load_and_fire(1, 1)

    def outer(g, carry):
        for b in range(2):
            c = 2 * g + b
            drain(b)
            compute(c, b)

            @pl.when(c + 2 < N_CHUNKS)
            def _():
                load_and_fire(c + 2, b)

        return carry

    lax.fori_loop(0, N_CHUNKS // 2, outer, 0)


def _mesh():
    return plsc.VectorSubcoreMesh(
        core_axis_name="c", subcore_axis_name="s", num_cores=NC, num_subcores=NS
    )


@jax.jit
def _emb(idx, tt, tail):
    tflat = pl.kernel(
        _tr_body,
        out_type=jax.ShapeDtypeStruct((VOCAB * EMBED,), jnp.float32),
        mesh=_mesh(),
        scratch_types=[
            pltpu.VMEM((EMBED, VB), jnp.float32),
            pltpu.VMEM((EMBED, VB), jnp.float32),
            pltpu.VMEM((BLK_ELS,), jnp.float32),
            pltpu.VMEM((BLK_ELS,), jnp.float32),
            pltpu.VMEM((TAIL * EMBED,), jnp.float32),
            pltpu.SemaphoreType.DMA,
            pltpu.SemaphoreType.DMA,
            pltpu.SemaphoreType.DMA,
            pltpu.SemaphoreType.DMA,
        ],
        compiler_params=pltpu.CompilerParams(
            use_tc_tiling_on_sc=True, needs_layout_passes=False
        ),
    )(tt, tail)

    table = tflat.reshape(VOCAB, EMBED)
    out = pl.kernel(
        _gather_body,
        out_type=jax.ShapeDtypeStruct((SEGS, EMBED), jnp.float32),
        mesh=_mesh(),
        scratch_types=[
            pltpu.VMEM((2, ROWS_PER_CHUNK), jnp.int32),
            pltpu.VMEM((2, ROWS_PER_CHUNK, EMBED), jnp.float32),
            pltpu.VMEM((2, CHUNK_SEGS, EMBED), jnp.float32),
            pltpu.SemaphoreType.DMA,
            pltpu.SemaphoreType.DMA,
        ],
        compiler_params=pltpu.CompilerParams(use_tc_tiling_on_sc=False),
    )(idx, table)
    return out


def kernel(input, table):
    idx = input.reshape(SEGS * L)
    tt = table.T
    tail = lax.slice(table, (NBLK * VB, 0), (VOCAB, EMBED)).reshape(TAIL * EMBED)
    out = _emb(idx, tt, tail)
    return out.reshape(B, F, EMBED)
